# baseline (speedup 1.0000x reference)
import functools

import jax
import jax.numpy as jnp
import numpy as np
from jax import lax
from jax.experimental import pallas as pl
from jax.experimental.pallas import tpu as pltpu
from jax.experimental.pallas import tpu_sc as plsc

D_MODEL = 64
MAX_LEN = 256
SEQ = 200

NW = 32
GRP = 128
CHUNK_GRPS = 5
CHUNK = GRP * CHUNK_GRPS
NLANE = 16
SCALE = float(np.sqrt(np.float32(D_MODEL)))


@functools.lru_cache(maxsize=1)
def _pe_np():
    pos = np.arange(MAX_LEN)[:, np.newaxis]
    i = np.arange(D_MODEL)[np.newaxis, :]
    angle_rates = 1 / np.power(10000, 2 * (i // 2) / np.float32(D_MODEL))
    angle_rads = pos * angle_rates
    pe = np.zeros((MAX_LEN, D_MODEL), dtype=np.float32)
    pe[:, 0::2] = np.sin(angle_rads[:, 0::2])
    pe[:, 1::2] = np.cos(angle_rads[:, 1::2])
    return pe[:SEQ]


def _sc_body(table_hbm, idx_hbm, pe_hbm, out_hbm, idx_v, pe_v, buf, sem):
    wid = lax.axis_index("s") * 2 + lax.axis_index("c")
    rows_per_w = idx_hbm.shape[1] * idx_hbm.shape[2]
    nchunk = rows_per_w // CHUNK
    base_row = wid * rows_per_w

    pltpu.sync_copy(idx_hbm.at[wid], idx_v)
    pltpu.sync_copy(pe_hbm, pe_v)

    def chunk_body(c, carry):
        copies = []
        for i in range(CHUNK_GRPS):
            copies.append(pltpu.async_copy(
                table_hbm.at[idx_v.at[c * CHUNK_GRPS + i]],
                buf.at[pl.ds(i * GRP, GRP)],
                sem,
            ))
        for cp in copies:
            cp.wait()

        chunk_phase = lax.rem(c * CHUNK, SEQ)

        def row_body(r, carry2):
            p = lax.rem(chunk_phase + r, SEQ)
            for j in range(D_MODEL // NLANE):
                sl = pl.ds(j * NLANE, NLANE)
                buf[r, sl] = buf[r, sl] * SCALE + pe_v[p, sl]
            return carry2

        lax.fori_loop(0, CHUNK, row_body, 0)

        pltpu.sync_copy(buf, out_hbm.at[pl.ds(base_row + c * CHUNK, CHUNK)])
        return carry

    lax.fori_loop(0, nchunk, chunk_body, 0)


def kernel(x, table):
    batch, seq = x.shape
    n = batch * seq
    rows_per_w = n // NW
    idx3 = x.reshape(NW, rows_per_w // GRP, GRP)
    pe = jnp.asarray(_pe_np())

    mesh = plsc.VectorSubcoreMesh(core_axis_name="c", subcore_axis_name="s")
    run = pl.kernel(
        _sc_body,
        mesh=mesh,
        out_type=jax.ShapeDtypeStruct((n, D_MODEL), jnp.float32),
        scratch_types=[
            pltpu.VMEM((rows_per_w // GRP, GRP), jnp.int32),
            pltpu.VMEM((SEQ, D_MODEL), jnp.float32),
            pltpu.VMEM((CHUNK, D_MODEL), jnp.float32),
            pltpu.SemaphoreType.DMA,
        ],
        compiler_params=pltpu.CompilerParams(use_tc_tiling_on_sc=False),
    )
    out = run(table, idx3, pe)
    return out.reshape(batch, seq, D_MODEL)
